# Initial kernel scaffold; baseline (speedup 1.0000x reference)
import jax
import jax.numpy as jnp
from jax import lax
from jax.experimental import pallas as pl
from jax.experimental.pallas import tpu as pltpu
from jax.experimental.pallas import tpu_sc as plsc

D = 128
B = 16384
NR = 32768
NI = 16384
NM = 16384
NC = 2
NS = 16
L = 16
HALF = B // NC
TRASH = HALF
ACC_ROWS = HALF + 8
REG_CHUNK = NR // NS
SC2_CHUNK = B // NS

f32 = jnp.float32
i32 = jnp.int32


def _leaky(x):
    return jnp.where(x > 0, x, 0.1 * x)


def _imm_pre(x):
    sign = jnp.sign(x)
    mod = jnp.abs(x)
    val = jnp.log2(mod) * sign
    return jnp.where(mod > 2, val, x)


def _sc1_body(table, mnemic, reg_tok, reg_idx, mem_tok, z128, z16, o16,
              mnem_g, memcat, sreg, cnt16,
              idxbuf, tokbuf, ctok, cloc, rowbuf, cbuf, onesbuf, acc, cnt):
    c = lax.axis_index("c")
    s = lax.axis_index("s")
    gid = c * NS + s
    lane = jnp.arange(L, dtype=i32)
    srow = pl.multiple_of(s * 512, 512)

    pltpu.sync_copy(z128, rowbuf.at[pl.ds(0, 128)])
    pltpu.sync_copy(z16, cbuf.at[pl.ds(0, 128)])
    pltpu.sync_copy(o16, onesbuf)
    for k in range(4):
        pltpu.sync_copy(rowbuf.at[pl.ds(0, 128)],
                        acc.at[pl.ds(srow + k * 128, 128)])
        pltpu.sync_copy(cbuf.at[pl.ds(0, 128)],
                        cnt.at[pl.ds(srow + k * 128, 128)])
    plsc.subcore_barrier()

    def pre(i, _):
        q = i * L + lane
        plsc.store_scatter(ctok, [q], jnp.zeros((L,), i32))
        plsc.store_scatter(cloc, [q >> 7, q & 127], jnp.full((L,), TRASH, i32))
        return 0

    lax.fori_loop(0, REG_CHUNK // L, pre, 0)

    base = pl.multiple_of(s * REG_CHUNK, REG_CHUNK)
    pltpu.sync_copy(reg_idx.at[pl.ds(base, REG_CHUNK)], idxbuf)
    pltpu.sync_copy(reg_tok.at[pl.ds(base, REG_CHUNK)], tokbuf)
    lo = c * HALF

    def compact(i, n):
        idx = idxbuf[pl.ds(i * L, L)]
        tok = tokbuf[pl.ds(i * L, L)]
        m = (idx >= lo) & (idx < lo + HALF)
        mi = m.astype(i32)
        p = n + plsc.cumsum(mi) - 1
        plsc.store_scatter(ctok, [p], tok, mask=m)
        plsc.store_scatter(cloc, [p >> 7, p & 127], idx - lo, mask=m)
        return n + jnp.sum(mi)

    nmatch = lax.fori_loop(0, REG_CHUNK // L, compact, jnp.int32(0))

    def gsc(g, _):
        off = pl.multiple_of(g * 128, 128)
        pltpu.sync_copy(table.at[ctok.at[pl.ds(off, 128)]],
                        rowbuf.at[pl.ds(0, 128)])
        pltpu.sync_copy(rowbuf.at[pl.ds(0, 128)], acc.at[cloc.at[g]], add=True)
        pltpu.sync_copy(onesbuf, cnt.at[cloc.at[g]], add=True)
        return 0

    lax.fori_loop(0, (nmatch + 127) // 128, gsc, 0)

    mbase = pl.multiple_of(gid * 512, 512)
    pltpu.sync_copy(mnemic.at[pl.ds(mbase, 512)], idxbuf.at[pl.ds(0, 512)])
    for k in range(4):
        pltpu.sync_copy(table.at[idxbuf.at[pl.ds(k * 128, 128)]],
                        rowbuf.at[pl.ds(k * 128, 128)])
    pltpu.sync_copy(rowbuf, mnem_g.at[pl.ds(mbase, 512)])

    tbase = pl.multiple_of(gid * 1536, 512)
    for r in range(3):
        pltpu.sync_copy(mem_tok.at[pl.ds(tbase + r * 512, 512)],
                        idxbuf.at[pl.ds(0, 512)])
        for k in range(4):
            pltpu.sync_copy(table.at[idxbuf.at[pl.ds(k * 128, 128)]],
                            rowbuf.at[pl.ds(k * 128, 128)])
        pltpu.sync_copy(rowbuf, memcat.at[pl.ds(tbase + r * 512, 512)])

    plsc.subcore_barrier()

    obase = pl.multiple_of(c * HALF + s * 512, 512)
    pltpu.sync_copy(acc.at[pl.ds(srow, 512)], rowbuf)
    pltpu.sync_copy(rowbuf, sreg.at[pl.ds(obase, 512)])
    pltpu.sync_copy(cnt.at[pl.ds(srow, 512)], cbuf)
    pltpu.sync_copy(cbuf, cnt16.at[pl.ds(obase, 512)])


_sc1 = pl.kernel(
    _sc1_body,
    out_type=(
        jax.ShapeDtypeStruct((B, D), f32),
        jax.ShapeDtypeStruct((3 * NM, D), f32),
        jax.ShapeDtypeStruct((B, D), f32),
        jax.ShapeDtypeStruct((B, 16), f32),
    ),
    mesh=plsc.VectorSubcoreMesh(core_axis_name="c", subcore_axis_name="s",
                                num_cores=NC, num_subcores=NS),
    scratch_types=(
        pltpu.VMEM((REG_CHUNK,), i32),
        pltpu.VMEM((REG_CHUNK,), i32),
        pltpu.VMEM((REG_CHUNK,), i32),
        pltpu.VMEM((REG_CHUNK // 128, 128), i32),
        pltpu.VMEM((512, D), f32),
        pltpu.VMEM((512, 16), f32),
        pltpu.VMEM((128, 16), f32),
        pltpu.VMEM_SHARED((ACC_ROWS, D), f32),
        pltpu.VMEM_SHARED((ACC_ROWS, 16), f32),
    ),
)


def _sc2_body(rm, ops1, ops2, idx_imm, idx_mem, m_out,
              idxbuf, ctok, cloc, rowbuf, acc):
    c = lax.axis_index("c")
    s = lax.axis_index("s")
    lane = jnp.arange(L, dtype=i32)
    srow = pl.multiple_of(s * 512, 512)
    obase = pl.multiple_of(c * HALF + s * 512, 512)
    lo = c * HALF

    pltpu.sync_copy(rm.at[pl.ds(obase, 512)], rowbuf)
    pltpu.sync_copy(rowbuf, acc.at[pl.ds(srow, 512)])
    plsc.subcore_barrier()

    def one_pass(idx_hbm, src_hbm):
        def pre(i, _):
            q = i * L + lane
            plsc.store_scatter(ctok, [q], jnp.zeros((L,), i32))
            plsc.store_scatter(cloc, [q >> 7, q & 127],
                               jnp.full((L,), TRASH, i32))
            return 0

        lax.fori_loop(0, SC2_CHUNK // L, pre, 0)

        base = pl.multiple_of(s * SC2_CHUNK, SC2_CHUNK)
        pltpu.sync_copy(idx_hbm.at[pl.ds(base, SC2_CHUNK)], idxbuf)

        def compact(i, n):
            idx = idxbuf[pl.ds(i * L, L)]
            m = (idx >= lo) & (idx < lo + HALF)
            mi = m.astype(i32)
            p = n + plsc.cumsum(mi) - 1
            plsc.store_scatter(ctok, [p], base + i * L + lane, mask=m)
            plsc.store_scatter(cloc, [p >> 7, p & 127], idx - lo, mask=m)
            return n + jnp.sum(mi)

        nmatch = lax.fori_loop(0, SC2_CHUNK // L, compact, jnp.int32(0))

        def gsc(g, _):
            off = pl.multiple_of(g * 128, 128)
            pltpu.sync_copy(src_hbm.at[ctok.at[pl.ds(off, 128)]],
                            rowbuf.at[pl.ds(0, 128)])
            pltpu.sync_copy(rowbuf.at[pl.ds(0, 128)], acc.at[cloc.at[g]],
                            add=True)
            return 0

        lax.fori_loop(0, (nmatch + 127) // 128, gsc, 0)

    one_pass(idx_imm, ops1)
    one_pass(idx_mem, ops2)
    plsc.subcore_barrier()

    pltpu.sync_copy(acc.at[pl.ds(srow, 512)], rowbuf)
    pltpu.sync_copy(rowbuf, m_out.at[pl.ds(obase, 512)])


_sc2 = pl.kernel(
    _sc2_body,
    out_type=jax.ShapeDtypeStruct((B, D), f32),
    mesh=plsc.VectorSubcoreMesh(core_axis_name="c", subcore_axis_name="s",
                                num_cores=NC, num_subcores=NS),
    scratch_types=(
        pltpu.VMEM((SC2_CHUNK,), i32),
        pltpu.VMEM((SC2_CHUNK,), i32),
        pltpu.VMEM((SC2_CHUNK // 128, 128), i32),
        pltpu.VMEM((512, D), f32),
        pltpu.VMEM_SHARED((ACC_ROWS, D), f32),
    ),
)


RB = 1024


def _tcb_body(imm_ref, disp_ref, memcat_ref, sreg_ref, cnt_ref, mnem_ref,
              iW1, ib1, iW2, ib2, mW1, mb1, mW2, mb2,
              oW0, ob0, oW1, ob1, oW2, ob2,
              ops1_ref, ops2_ref, r_ref):
    w1i = iW1[...]
    b1i = ib1[...][None, :]
    w2i = iW2[...]
    b2i = ib2[...][None, :]

    x = _imm_pre(imm_ref[...])
    h = _leaky(x * w1i + b1i)
    imm_e = _leaky(jnp.dot(h, w2i, preferred_element_type=f32) + b2i)
    ops1_ref[...] = jnp.dot(imm_e, oW1[...], preferred_element_type=f32) \
        + ob1[...][None, :]

    dx = _imm_pre(disp_ref[...])
    hd = _leaky(dx * w1i + b1i)
    disp_e = _leaky(jnp.dot(hd, w2i, preferred_element_type=f32) + b2i)

    w1m = mW1[...]
    m1 = (jnp.dot(memcat_ref[...], w1m[:3 * D], preferred_element_type=f32)
          + jnp.dot(disp_e, w1m[3 * D:], preferred_element_type=f32)
          + mb1[...][None, :])
    mem_e = _leaky(jnp.dot(_leaky(m1), mW2[...], preferred_element_type=f32)
                   + mb2[...][None, :])
    ops2_ref[...] = jnp.dot(mem_e, oW2[...], preferred_element_type=f32) \
        + ob2[...][None, :]

    r_ref[...] = (mnem_ref[...]
                  + jnp.dot(sreg_ref[...], oW0[...], preferred_element_type=f32)
                  + cnt_ref[:, 0:1] * ob0[...][None, :])


def _row_spec(cols):
    return pl.BlockSpec((RB, cols), lambda i: (i, 0))


def _full2(shape):
    return pl.BlockSpec(shape, lambda i: (0, 0))


def _full1(n):
    return pl.BlockSpec((n,), lambda i: (0,))


_tcb = pl.pallas_call(
    _tcb_body,
    grid=(B // RB,),
    in_specs=[
        _row_spec(1), _row_spec(1), _row_spec(3 * D), _row_spec(D),
        _row_spec(16), _row_spec(D),
        _full2((1, D)), _full1(D), _full2((D, D)), _full1(D),
        _full2((4 * D, D)), _full1(D), _full2((D, D)), _full1(D),
        _full2((D, D)), _full1(D), _full2((D, D)), _full1(D),
        _full2((D, D)), _full1(D),
    ],
    out_specs=[_row_spec(D)] * 3,
    out_shape=[jax.ShapeDtypeStruct((B, D), f32)] * 3,
)


def _tcd_body(m_ref, aggW_ref, aggb_ref, out_ref):
    out_ref[...] = _leaky(jnp.dot(_leaky(m_ref[...]), aggW_ref[...],
                                  preferred_element_type=f32)
                          + aggb_ref[...][None, :])


_tcd = pl.pallas_call(
    _tcd_body,
    grid=(B // RB,),
    in_specs=[_row_spec(D), _full2((D, D)), _full1(D)],
    out_specs=_row_spec(D),
    out_shape=jax.ShapeDtypeStruct((B, D), f32),
)


def kernel(mnemic, reg_tokens, imm_vals, mem_tokens, mem_disp,
           ins_idx_reg, ins_idx_imm, ins_idx_mem,
           table, imm_W1, imm_b1, imm_W2, imm_b2,
           mem_W1, mem_b1, mem_W2, mem_b2,
           opW0, opb0, opW1, opb1, opW2, opb2, aggW, aggb):
    mnemic = mnemic.astype(i32)
    reg_tokens = reg_tokens.astype(i32)
    mem_flat = mem_tokens.astype(i32).reshape(-1)
    iir = ins_idx_reg.astype(i32)
    iii = ins_idx_imm.astype(i32)
    iim = ins_idx_mem.astype(i32)
    z128 = jnp.zeros((128, D), f32)
    z16 = jnp.zeros((128, 16), f32)
    o16 = jnp.ones((128, 16), f32)

    mnem_g, memcat, sreg, cnt16 = _sc1(table, mnemic, reg_tokens, iir,
                                       mem_flat, z128, z16, o16)
    ops1, ops2, r = _tcb(imm_vals, mem_disp.reshape(NM, 1),
                         memcat.reshape(NM, 3 * D), sreg, cnt16, mnem_g,
                         imm_W1, imm_b1, imm_W2, imm_b2,
                         mem_W1, mem_b1, mem_W2, mem_b2,
                         opW0, opb0, opW1, opb1, opW2, opb2)
    m = _sc2(r, ops1, ops2, iii, iim)
    return _tcd(m, aggW, aggb)


# speedup vs baseline: 1.1947x; 1.1947x over previous
import jax
import jax.numpy as jnp
from jax import lax
from jax.experimental import pallas as pl
from jax.experimental.pallas import tpu as pltpu
from jax.experimental.pallas import tpu_sc as plsc

D = 128
B = 16384
NR = 32768
NI = 16384
NM = 16384
NC = 2
NS = 16
L = 16
NPASS = 2
Q = B // (NC * NPASS)
QT = Q // NS
TRASH = Q
ACC_ROWS = Q + 8
REG_CHUNK = NR // NS
SC2_CHUNK = B // NS

f32 = jnp.float32
i32 = jnp.int32


def _leaky(x):
    return jnp.where(x > 0, x, 0.1 * x)


def _imm_pre(x):
    sign = jnp.sign(x)
    mod = jnp.abs(x)
    val = jnp.log2(mod) * sign
    return jnp.where(mod > 2, val, x)


def _prefill(ctok, cloc, nvec, lane):
    def pre(i, _):
        q = i * L + lane
        plsc.store_scatter(ctok, [q], jnp.zeros((L,), i32))
        plsc.store_scatter(cloc, [q >> 7, q & 127], jnp.full((L,), TRASH, i32))
        return 0
    lax.fori_loop(0, nvec, pre, 0)


def _sc1_body(table, mnemic, reg_tok, reg_idx, mem_tok, z128, o128,
              mnem_g, memcat, sreg, cnt,
              idxbuf, tokbuf, ctok, cloc, rowbuf, onesbuf, acc):
    c = lax.axis_index("c")
    s = lax.axis_index("s")
    gid = c * NS + s
    lane = jnp.arange(L, dtype=i32)

    pltpu.sync_copy(o128, onesbuf)

    mbase = pl.multiple_of(gid * 512, 512)
    pltpu.sync_copy(mnemic.at[pl.ds(mbase, 512)], idxbuf.at[pl.ds(0, 512)])
    for k in range(4):
        pltpu.sync_copy(table.at[idxbuf.at[pl.ds(k * 128, 128)]], rowbuf)
        pltpu.sync_copy(rowbuf, mnem_g.at[pl.ds(mbase + k * 128, 128)])

    tbase = pl.multiple_of(gid * 1536, 512)
    pltpu.sync_copy(mem_tok.at[pl.ds(tbase, 1536)], idxbuf.at[pl.ds(0, 1536)])
    for k in range(12):
        pltpu.sync_copy(table.at[idxbuf.at[pl.ds(k * 128, 128)]], rowbuf)
        pltpu.sync_copy(rowbuf, memcat.at[pl.ds(tbase + k * 128, 128)])

    base = pl.multiple_of(s * REG_CHUNK, REG_CHUNK)
    pltpu.sync_copy(reg_idx.at[pl.ds(base, REG_CHUNK)], idxbuf)
    pltpu.sync_copy(reg_tok.at[pl.ds(base, REG_CHUNK)], tokbuf)
    srow = pl.multiple_of(s * QT, QT)

    for q in range(NPASS):
        lo = c * (NPASS * Q) + q * Q
        obase = pl.multiple_of(lo + s * QT, QT)

        pltpu.sync_copy(z128, rowbuf)
        for k in range(QT // 128):
            pltpu.sync_copy(rowbuf, acc.at[pl.ds(srow + k * 128, 128)])
        plsc.subcore_barrier()

        _prefill(ctok, cloc, REG_CHUNK // L, lane)

        def compact(i, n):
            idx = idxbuf[pl.ds(i * L, L)]
            tok = tokbuf[pl.ds(i * L, L)]
            m = (idx >= lo) & (idx < lo + Q)
            mi = m.astype(i32)
            p = n + plsc.cumsum(mi) - 1
            plsc.store_scatter(ctok, [p], tok, mask=m)
            plsc.store_scatter(cloc, [p >> 7, p & 127], idx - lo, mask=m)
            return n + jnp.sum(mi)

        nmatch = lax.fori_loop(0, REG_CHUNK // L, compact, jnp.int32(0))

        for g in range(REG_CHUNK // 128):
            @pl.when(g * 128 < nmatch)
            def _():
                pltpu.sync_copy(table.at[ctok.at[pl.ds(g * 128, 128)]],
                                rowbuf)
                pltpu.sync_copy(rowbuf, acc.at[cloc.at[g]], add=True)

        plsc.subcore_barrier()
        for k in range(QT // 128):
            pltpu.sync_copy(acc.at[pl.ds(srow + k * 128, 128)], rowbuf)
            pltpu.sync_copy(rowbuf, sreg.at[pl.ds(obase + k * 128, 128)])

        pltpu.sync_copy(z128, rowbuf)
        for k in range(QT // 128):
            pltpu.sync_copy(rowbuf, acc.at[pl.ds(srow + k * 128, 128)])
        plsc.subcore_barrier()

        for g in range(REG_CHUNK // 128):
            @pl.when(g * 128 < nmatch)
            def _():
                pltpu.sync_copy(onesbuf, acc.at[cloc.at[g]], add=True)

        plsc.subcore_barrier()
        for k in range(QT // 128):
            pltpu.sync_copy(acc.at[pl.ds(srow + k * 128, 128)], rowbuf)
            pltpu.sync_copy(rowbuf, cnt.at[pl.ds(obase + k * 128, 128)])


_sc1 = pl.kernel(
    _sc1_body,
    out_type=(
        jax.ShapeDtypeStruct((B, D), f32),
        jax.ShapeDtypeStruct((3 * NM, D), f32),
        jax.ShapeDtypeStruct((B, D), f32),
        jax.ShapeDtypeStruct((B, D), f32),
    ),
    mesh=plsc.VectorSubcoreMesh(core_axis_name="c", subcore_axis_name="s",
                                num_cores=NC, num_subcores=NS),
    scratch_types=(
        pltpu.VMEM((REG_CHUNK,), i32),
        pltpu.VMEM((REG_CHUNK,), i32),
        pltpu.VMEM((REG_CHUNK,), i32),
        pltpu.VMEM((REG_CHUNK // 128, 128), i32),
        pltpu.VMEM((128, D), f32),
        pltpu.VMEM((128, D), f32),
        pltpu.VMEM_SHARED((ACC_ROWS, D), f32),
    ),
    compiler_params=pltpu.CompilerParams(needs_layout_passes=False),
)


def _sc2_body(rm, ops1, ops2, idx_imm, idx_mem, m_out,
              idxbuf, ctok, cloc, rowbuf, acc):
    c = lax.axis_index("c")
    s = lax.axis_index("s")
    lane = jnp.arange(L, dtype=i32)
    srow = pl.multiple_of(s * QT, QT)
    base = pl.multiple_of(s * SC2_CHUNK, SC2_CHUNK)

    for q in range(NPASS):
        lo = c * (NPASS * Q) + q * Q
        obase = pl.multiple_of(lo + s * QT, QT)
        for k in range(QT // 128):
            pltpu.sync_copy(rm.at[pl.ds(obase + k * 128, 128)], rowbuf)
            pltpu.sync_copy(rowbuf, acc.at[pl.ds(srow + k * 128, 128)])
        plsc.subcore_barrier()

        for idx_hbm, src_hbm in ((idx_imm, ops1), (idx_mem, ops2)):
            _prefill(ctok, cloc, SC2_CHUNK // L, lane)
            pltpu.sync_copy(idx_hbm.at[pl.ds(base, SC2_CHUNK)], idxbuf)

            def compact(i, n):
                idx = idxbuf[pl.ds(i * L, L)]
                m = (idx >= lo) & (idx < lo + Q)
                mi = m.astype(i32)
                p = n + plsc.cumsum(mi) - 1
                plsc.store_scatter(ctok, [p], base + i * L + lane, mask=m)
                plsc.store_scatter(cloc, [p >> 7, p & 127], idx - lo, mask=m)
                return n + jnp.sum(mi)

            nmatch = lax.fori_loop(0, SC2_CHUNK // L, compact, jnp.int32(0))

            for g in range(SC2_CHUNK // 128):
                @pl.when(g * 128 < nmatch)
                def _():
                    pltpu.sync_copy(src_hbm.at[ctok.at[pl.ds(g * 128, 128)]],
                                    rowbuf)
                    pltpu.sync_copy(rowbuf, acc.at[cloc.at[g]], add=True)

        plsc.subcore_barrier()
        for k in range(QT // 128):
            pltpu.sync_copy(acc.at[pl.ds(srow + k * 128, 128)], rowbuf)
            pltpu.sync_copy(rowbuf, m_out.at[pl.ds(obase + k * 128, 128)])


_sc2 = pl.kernel(
    _sc2_body,
    out_type=jax.ShapeDtypeStruct((B, D), f32),
    mesh=plsc.VectorSubcoreMesh(core_axis_name="c", subcore_axis_name="s",
                                num_cores=NC, num_subcores=NS),
    scratch_types=(
        pltpu.VMEM((SC2_CHUNK,), i32),
        pltpu.VMEM((SC2_CHUNK,), i32),
        pltpu.VMEM((SC2_CHUNK // 128, 128), i32),
        pltpu.VMEM((128, D), f32),
        pltpu.VMEM_SHARED((ACC_ROWS, D), f32),
    ),
    compiler_params=pltpu.CompilerParams(needs_layout_passes=False),
)


RB = 1024


def _tcb_body(imm_ref, disp_ref, memcat_ref, sreg_ref, cnt_ref, mnem_ref,
              iW1, ib1, iW2, ib2, mW1, mb1, mW2, mb2,
              oW0, ob0, oW1, ob1, oW2, ob2,
              ops1_ref, ops2_ref, r_ref):
    w1i = iW1[...]
    b1i = ib1[...][None, :]
    w2i = iW2[...]
    b2i = ib2[...][None, :]

    x = _imm_pre(imm_ref[...])
    h = _leaky(x * w1i + b1i)
    imm_e = _leaky(jnp.dot(h, w2i, preferred_element_type=f32) + b2i)
    ops1_ref[...] = jnp.dot(imm_e, oW1[...], preferred_element_type=f32) \
        + ob1[...][None, :]

    dx = _imm_pre(disp_ref[...])
    hd = _leaky(dx * w1i + b1i)
    disp_e = _leaky(jnp.dot(hd, w2i, preferred_element_type=f32) + b2i)

    w1m = mW1[...]
    m1 = (jnp.dot(memcat_ref[...], w1m[:3 * D], preferred_element_type=f32)
          + jnp.dot(disp_e, w1m[3 * D:], preferred_element_type=f32)
          + mb1[...][None, :])
    mem_e = _leaky(jnp.dot(_leaky(m1), mW2[...], preferred_element_type=f32)
                   + mb2[...][None, :])
    ops2_ref[...] = jnp.dot(mem_e, oW2[...], preferred_element_type=f32) \
        + ob2[...][None, :]

    r_ref[...] = (mnem_ref[...]
                  + jnp.dot(sreg_ref[...], oW0[...], preferred_element_type=f32)
                  + cnt_ref[:, 0:1] * ob0[...][None, :])


def _row_spec(cols):
    return pl.BlockSpec((RB, cols), lambda i: (i, 0))


def _full2(shape):
    return pl.BlockSpec(shape, lambda i: (0, 0))


def _full1(n):
    return pl.BlockSpec((n,), lambda i: (0,))


_tcb = pl.pallas_call(
    _tcb_body,
    grid=(B // RB,),
    in_specs=[
        _row_spec(1), _row_spec(1), _row_spec(3 * D), _row_spec(D),
        _row_spec(D), _row_spec(D),
        _full2((1, D)), _full1(D), _full2((D, D)), _full1(D),
        _full2((4 * D, D)), _full1(D), _full2((D, D)), _full1(D),
        _full2((D, D)), _full1(D), _full2((D, D)), _full1(D),
        _full2((D, D)), _full1(D),
    ],
    out_specs=[_row_spec(D)] * 3,
    out_shape=[jax.ShapeDtypeStruct((B, D), f32)] * 3,
)


def _tcd_body(m_ref, aggW_ref, aggb_ref, out_ref):
    out_ref[...] = _leaky(jnp.dot(_leaky(m_ref[...]), aggW_ref[...],
                                  preferred_element_type=f32)
                          + aggb_ref[...][None, :])


_tcd = pl.pallas_call(
    _tcd_body,
    grid=(B // RB,),
    in_specs=[_row_spec(D), _full2((D, D)), _full1(D)],
    out_specs=_row_spec(D),
    out_shape=jax.ShapeDtypeStruct((B, D), f32),
)


def kernel(mnemic, reg_tokens, imm_vals, mem_tokens, mem_disp,
           ins_idx_reg, ins_idx_imm, ins_idx_mem,
           table, imm_W1, imm_b1, imm_W2, imm_b2,
           mem_W1, mem_b1, mem_W2, mem_b2,
           opW0, opb0, opW1, opb1, opW2, opb2, aggW, aggb):
    mnemic = mnemic.astype(i32)
    reg_tokens = reg_tokens.astype(i32)
    mem_flat = mem_tokens.astype(i32).reshape(-1)
    iir = ins_idx_reg.astype(i32)
    iii = ins_idx_imm.astype(i32)
    iim = ins_idx_mem.astype(i32)
    z128 = jnp.zeros((128, D), f32)
    o128 = jnp.ones((128, D), f32)

    mnem_g, memcat, sreg, cnt = _sc1(table, mnemic, reg_tokens, iir,
                                     mem_flat, z128, o128)
    ops1, ops2, r = _tcb(imm_vals, mem_disp.reshape(NM, 1),
                         memcat.reshape(NM, 3 * D), sreg, cnt, mnem_g,
                         imm_W1, imm_b1, imm_W2, imm_b2,
                         mem_W1, mem_b1, mem_W2, mem_b2,
                         opW0, opb0, opW1, opb1, opW2, opb2)
    m = _sc2(r, ops1, ops2, iii, iim)
    return _tcd(m, aggW, aggb)


# speedup vs baseline: 1.2489x; 1.0453x over previous
import jax
import jax.numpy as jnp
from jax import lax
from jax.experimental import pallas as pl
from jax.experimental.pallas import tpu as pltpu
from jax.experimental.pallas import tpu_sc as plsc

D = 128
B = 16384
NR = 32768
NI = 16384
NM = 16384
NC = 2
NS = 16
L = 16
NPASS = 2
Q = B // (NC * NPASS)
QT = Q // NS
TRASH = Q
ACC_ROWS = Q + 8
REG_CHUNK = NR // NS
SC2_CHUNK = B // NS

f32 = jnp.float32
i32 = jnp.int32


def _leaky(x):
    return jnp.where(x > 0, x, 0.1 * x)


def _imm_pre(x):
    sign = jnp.sign(x)
    mod = jnp.abs(x)
    val = jnp.log2(mod) * sign
    return jnp.where(mod > 2, val, x)


def _compact(idxbuf, tokvals, ctok, cloc, lo, hi, nvec, lane):
    def step(i, n):
        idx = idxbuf[pl.ds(i * L, L)]
        m = (idx >= lo) & (idx < hi)
        mi = m.astype(i32)
        p = n + plsc.cumsum(mi) - 1
        plsc.store_scatter(ctok, [p], tokvals(i), mask=m)
        plsc.store_scatter(cloc, [p >> 7, p & 127], idx - lo, mask=m)
        return n + jnp.sum(mi)

    nmatch = lax.fori_loop(0, nvec, step, jnp.int32(0))
    ceil_ = ((nmatch + 127) >> 7) << 7
    for j in range(8):
        p = nmatch + j * L + lane
        m = p < ceil_
        plsc.store_scatter(ctok, [p], jnp.zeros((L,), i32), mask=m)
        plsc.store_scatter(cloc, [p >> 7, p & 127],
                           jnp.full((L,), TRASH, i32), mask=m)
    return nmatch


def _pipe_gather_scatter(src_hbm, ctok, cloc, bufs, gsems, ssems, acc,
                         nmatch, nch):
    @pl.when(nmatch > 0)
    def _():
        pltpu.async_copy(src_hbm.at[ctok.at[pl.ds(0, 128)]], bufs[0],
                         gsems[0])

    for g in range(nch):
        b = g & 1
        nb = (g + 1) & 1

        @pl.when(g * 128 < nmatch)
        def _():
            if g >= 1:
                pltpu.make_async_copy(bufs[nb], acc.at[cloc.at[g - 1]],
                                      ssems[nb]).wait()
            if g + 1 < nch:
                @pl.when((g + 1) * 128 < nmatch)
                def _():
                    off = pl.multiple_of((g + 1) * 128, 128)
                    pltpu.async_copy(src_hbm.at[ctok.at[pl.ds(off, 128)]],
                                     bufs[nb], gsems[nb])
            pltpu.make_async_copy(src_hbm.at[ctok.at[pl.ds(g * 128, 128)]],
                                  bufs[b], gsems[b]).wait()
            pltpu.async_copy(bufs[b], acc.at[cloc.at[g]], ssems[b], add=True)

    last_par = ((nmatch + 127) >> 7) & 1

    @pl.when((nmatch > 0) & (last_par == 1))
    def _():
        pltpu.make_async_copy(bufs[0], acc.at[cloc.at[0]], ssems[0]).wait()

    @pl.when((nmatch > 0) & (last_par == 0))
    def _():
        pltpu.make_async_copy(bufs[1], acc.at[cloc.at[0]], ssems[1]).wait()


def _pipe_gather_out(table, idxbuf, out_hbm, out_base, nch, bufs, gsems,
                     wsems):
    pltpu.async_copy(table.at[idxbuf.at[pl.ds(0, 128)]], bufs[0], gsems[0])
    for k in range(nch):
        b = k & 1
        nb = (k + 1) & 1
        if k >= 1:
            pltpu.make_async_copy(
                bufs[nb], out_hbm.at[pl.ds(out_base, 128)], wsems[nb]).wait()
        if k + 1 < nch:
            pltpu.async_copy(
                table.at[idxbuf.at[pl.ds((k + 1) * 128, 128)]], bufs[nb],
                gsems[nb])
        pltpu.make_async_copy(table.at[idxbuf.at[pl.ds(k * 128, 128)]],
                              bufs[b], gsems[b]).wait()
        pltpu.async_copy(bufs[b], out_hbm.at[pl.ds(out_base + k * 128, 128)],
                         wsems[b])
    pltpu.make_async_copy(bufs[(nch - 1) & 1],
                          out_hbm.at[pl.ds(out_base, 128)],
                          wsems[(nch - 1) & 1]).wait()


def _copy2(src0, src1, dst0, dst1, bufs, gsems, ssems):
    pltpu.async_copy(src0, bufs[0], gsems[0])
    pltpu.async_copy(src1, bufs[1], gsems[1])
    pltpu.make_async_copy(src0, bufs[0], gsems[0]).wait()
    pltpu.async_copy(bufs[0], dst0, ssems[0])
    pltpu.make_async_copy(src1, bufs[1], gsems[1]).wait()
    pltpu.async_copy(bufs[1], dst1, ssems[1])
    pltpu.make_async_copy(bufs[0], dst0, ssems[0]).wait()
    pltpu.make_async_copy(bufs[1], dst1, ssems[1]).wait()


def _sc1_body(table, mnemic, reg_tok, reg_idx, mem_tok, z128, o128,
              mnem_g, memcat, sreg, cnt,
              idxbuf, tokbuf, ctok, cloc, bufA, bufB,
              gsemA, gsemB, ssemA, ssemB, acc):
    c = lax.axis_index("c")
    s = lax.axis_index("s")
    gid = c * NS + s
    lane = jnp.arange(L, dtype=i32)
    bufs = (bufA, bufB)
    gsems = (gsemA, gsemB)
    ssems = (ssemA, ssemB)

    mbase = pl.multiple_of(gid * 512, 512)
    pltpu.sync_copy(mnemic.at[pl.ds(mbase, 512)], idxbuf.at[pl.ds(0, 512)])
    _pipe_gather_out(table, idxbuf, mnem_g, mbase, 4, bufs, gsems, ssems)

    tbase = pl.multiple_of(gid * 1536, 512)
    pltpu.sync_copy(mem_tok.at[pl.ds(tbase, 1536)], idxbuf.at[pl.ds(0, 1536)])
    _pipe_gather_out(table, idxbuf, memcat, tbase, 12, bufs, gsems, ssems)

    base = pl.multiple_of(s * REG_CHUNK, REG_CHUNK)
    pltpu.sync_copy(reg_idx.at[pl.ds(base, REG_CHUNK)], idxbuf)
    pltpu.sync_copy(reg_tok.at[pl.ds(base, REG_CHUNK)], tokbuf)
    srow = pl.multiple_of(s * QT, QT)

    for q in range(NPASS):
        lo = c * (NPASS * Q) + q * Q
        obase = pl.multiple_of(lo + s * QT, QT)

        pltpu.sync_copy(z128, bufA)
        pltpu.async_copy(bufA, acc.at[pl.ds(srow, 128)], ssemA)
        pltpu.async_copy(bufA, acc.at[pl.ds(srow + 128, 128)], ssemB)
        nmatch = _compact(idxbuf, lambda i: tokbuf[pl.ds(i * L, L)],
                          ctok, cloc, lo, lo + Q, REG_CHUNK // L, lane)
        pltpu.make_async_copy(bufA, acc.at[pl.ds(srow, 128)], ssemA).wait()
        pltpu.make_async_copy(bufA, acc.at[pl.ds(srow, 128)], ssemB).wait()
        plsc.subcore_barrier()

        _pipe_gather_scatter(table, ctok, cloc, bufs, gsems, ssems, acc,
                             nmatch, REG_CHUNK // 128)
        plsc.subcore_barrier()
        _copy2(acc.at[pl.ds(srow, 128)], acc.at[pl.ds(srow + 128, 128)],
               sreg.at[pl.ds(obase, 128)], sreg.at[pl.ds(obase + 128, 128)],
               bufs, gsems, ssems)

        pltpu.sync_copy(z128, bufA)
        pltpu.async_copy(bufA, acc.at[pl.ds(srow, 128)], ssemA)
        pltpu.async_copy(bufA, acc.at[pl.ds(srow + 128, 128)], ssemB)
        pltpu.make_async_copy(bufA, acc.at[pl.ds(srow, 128)], ssemA).wait()
        pltpu.make_async_copy(bufA, acc.at[pl.ds(srow, 128)], ssemB).wait()
        pltpu.sync_copy(o128, bufB)
        plsc.subcore_barrier()

        for g in range(REG_CHUNK // 128):
            @pl.when(g * 128 < nmatch)
            def _():
                pltpu.async_copy(bufB, acc.at[cloc.at[g]], ssemA, add=True)
        for g in range(REG_CHUNK // 128):
            @pl.when(g * 128 < nmatch)
            def _():
                pltpu.make_async_copy(bufB, acc.at[cloc.at[0]], ssemA).wait()

        plsc.subcore_barrier()
        _copy2(acc.at[pl.ds(srow, 128)], acc.at[pl.ds(srow + 128, 128)],
               cnt.at[pl.ds(obase, 128)], cnt.at[pl.ds(obase + 128, 128)],
               bufs, gsems, ssems)


_sc1 = pl.kernel(
    _sc1_body,
    out_type=(
        jax.ShapeDtypeStruct((B, D), f32),
        jax.ShapeDtypeStruct((3 * NM, D), f32),
        jax.ShapeDtypeStruct((B, D), f32),
        jax.ShapeDtypeStruct((B, D), f32),
    ),
    mesh=plsc.VectorSubcoreMesh(core_axis_name="c", subcore_axis_name="s",
                                num_cores=NC, num_subcores=NS),
    scratch_types=(
        pltpu.VMEM((REG_CHUNK,), i32),
        pltpu.VMEM((REG_CHUNK,), i32),
        pltpu.VMEM((REG_CHUNK,), i32),
        pltpu.VMEM((REG_CHUNK // 128, 128), i32),
        pltpu.VMEM((128, D), f32),
        pltpu.VMEM((128, D), f32),
        pltpu.SemaphoreType.DMA,
        pltpu.SemaphoreType.DMA,
        pltpu.SemaphoreType.DMA,
        pltpu.SemaphoreType.DMA,
        pltpu.VMEM_SHARED((ACC_ROWS, D), f32),
    ),
    compiler_params=pltpu.CompilerParams(needs_layout_passes=False),
)


def _sc2_body(rm, ops1, ops2, idx_imm, idx_mem, m_out,
              idxbuf, ctok, cloc, bufA, bufB,
              gsemA, gsemB, ssemA, ssemB, acc):
    c = lax.axis_index("c")
    s = lax.axis_index("s")
    lane = jnp.arange(L, dtype=i32)
    srow = pl.multiple_of(s * QT, QT)
    base = pl.multiple_of(s * SC2_CHUNK, SC2_CHUNK)
    bufs = (bufA, bufB)
    gsems = (gsemA, gsemB)
    ssems = (ssemA, ssemB)

    for q in range(NPASS):
        lo = c * (NPASS * Q) + q * Q
        obase = pl.multiple_of(lo + s * QT, QT)
        _copy2(rm.at[pl.ds(obase, 128)], rm.at[pl.ds(obase + 128, 128)],
               acc.at[pl.ds(srow, 128)], acc.at[pl.ds(srow + 128, 128)],
               bufs, gsems, ssems)
        plsc.subcore_barrier()

        for idx_hbm, src_hbm in ((idx_imm, ops1), (idx_mem, ops2)):
            pltpu.sync_copy(idx_hbm.at[pl.ds(base, SC2_CHUNK)], idxbuf)
            nmatch = _compact(idxbuf, lambda i: base + i * L + lane,
                              ctok, cloc, lo, lo + Q, SC2_CHUNK // L, lane)
            _pipe_gather_scatter(src_hbm, ctok, cloc, bufs, gsems, ssems,
                                 acc, nmatch, SC2_CHUNK // 128)

        plsc.subcore_barrier()
        _copy2(acc.at[pl.ds(srow, 128)], acc.at[pl.ds(srow + 128, 128)],
               m_out.at[pl.ds(obase, 128)], m_out.at[pl.ds(obase + 128, 128)],
               bufs, gsems, ssems)


_sc2 = pl.kernel(
    _sc2_body,
    out_type=jax.ShapeDtypeStruct((B, D), f32),
    mesh=plsc.VectorSubcoreMesh(core_axis_name="c", subcore_axis_name="s",
                                num_cores=NC, num_subcores=NS),
    scratch_types=(
        pltpu.VMEM((SC2_CHUNK,), i32),
        pltpu.VMEM((SC2_CHUNK,), i32),
        pltpu.VMEM((SC2_CHUNK // 128, 128), i32),
        pltpu.VMEM((128, D), f32),
        pltpu.VMEM((128, D), f32),
        pltpu.SemaphoreType.DMA,
        pltpu.SemaphoreType.DMA,
        pltpu.SemaphoreType.DMA,
        pltpu.SemaphoreType.DMA,
        pltpu.VMEM_SHARED((ACC_ROWS, D), f32),
    ),
    compiler_params=pltpu.CompilerParams(needs_layout_passes=False),
)


RB = 1024


def _tcb_body(imm_ref, disp_ref, memcat_ref, sreg_ref, cnt_ref, mnem_ref,
              iW1, ib1, iW2, ib2, mW1, mb1, mW2, mb2,
              oW0, ob0, oW1, ob1, oW2, ob2,
              ops1_ref, ops2_ref, r_ref):
    w1i = iW1[...]
    b1i = ib1[...][None, :]
    w2i = iW2[...]
    b2i = ib2[...][None, :]

    x = _imm_pre(imm_ref[...])
    h = _leaky(x * w1i + b1i)
    imm_e = _leaky(jnp.dot(h, w2i, preferred_element_type=f32) + b2i)
    ops1_ref[...] = jnp.dot(imm_e, oW1[...], preferred_element_type=f32) \
        + ob1[...][None, :]

    dx = _imm_pre(disp_ref[...])
    hd = _leaky(dx * w1i + b1i)
    disp_e = _leaky(jnp.dot(hd, w2i, preferred_element_type=f32) + b2i)

    w1m = mW1[...]
    m1 = (jnp.dot(memcat_ref[...], w1m[:3 * D], preferred_element_type=f32)
          + jnp.dot(disp_e, w1m[3 * D:], preferred_element_type=f32)
          + mb1[...][None, :])
    mem_e = _leaky(jnp.dot(_leaky(m1), mW2[...], preferred_element_type=f32)
                   + mb2[...][None, :])
    ops2_ref[...] = jnp.dot(mem_e, oW2[...], preferred_element_type=f32) \
        + ob2[...][None, :]

    r_ref[...] = (mnem_ref[...]
                  + jnp.dot(sreg_ref[...], oW0[...], preferred_element_type=f32)
                  + cnt_ref[:, 0:1] * ob0[...][None, :])


def _row_spec(cols):
    return pl.BlockSpec((RB, cols), lambda i: (i, 0))


def _full2(shape):
    return pl.BlockSpec(shape, lambda i: (0, 0))


def _full1(n):
    return pl.BlockSpec((n,), lambda i: (0,))


_tcb = pl.pallas_call(
    _tcb_body,
    grid=(B // RB,),
    in_specs=[
        _row_spec(1), _row_spec(1), _row_spec(3 * D), _row_spec(D),
        _row_spec(D), _row_spec(D),
        _full2((1, D)), _full1(D), _full2((D, D)), _full1(D),
        _full2((4 * D, D)), _full1(D), _full2((D, D)), _full1(D),
        _full2((D, D)), _full1(D), _full2((D, D)), _full1(D),
        _full2((D, D)), _full1(D),
    ],
    out_specs=[_row_spec(D)] * 3,
    out_shape=[jax.ShapeDtypeStruct((B, D), f32)] * 3,
)


def _tcd_body(m_ref, aggW_ref, aggb_ref, out_ref):
    out_ref[...] = _leaky(jnp.dot(_leaky(m_ref[...]), aggW_ref[...],
                                  preferred_element_type=f32)
                          + aggb_ref[...][None, :])


_tcd = pl.pallas_call(
    _tcd_body,
    grid=(B // RB,),
    in_specs=[_row_spec(D), _full2((D, D)), _full1(D)],
    out_specs=_row_spec(D),
    out_shape=jax.ShapeDtypeStruct((B, D), f32),
)


def kernel(mnemic, reg_tokens, imm_vals, mem_tokens, mem_disp,
           ins_idx_reg, ins_idx_imm, ins_idx_mem,
           table, imm_W1, imm_b1, imm_W2, imm_b2,
           mem_W1, mem_b1, mem_W2, mem_b2,
           opW0, opb0, opW1, opb1, opW2, opb2, aggW, aggb):
    mnemic = mnemic.astype(i32)
    reg_tokens = reg_tokens.astype(i32)
    mem_flat = mem_tokens.astype(i32).reshape(-1)
    iir = ins_idx_reg.astype(i32)
    iii = ins_idx_imm.astype(i32)
    iim = ins_idx_mem.astype(i32)
    z128 = jnp.zeros((128, D), f32)
    o128 = jnp.ones((128, D), f32)

    mnem_g, memcat, sreg, cnt = _sc1(table, mnemic, reg_tokens, iir,
                                     mem_flat, z128, o128)
    ops1, ops2, r = _tcb(imm_vals, mem_disp.reshape(NM, 1),
                         memcat.reshape(NM, 3 * D), sreg, cnt, mnem_g,
                         imm_W1, imm_b1, imm_W2, imm_b2,
                         mem_W1, mem_b1, mem_W2, mem_b2,
                         opW0, opb0, opW1, opb1, opW2, opb2)
    m = _sc2(r, ops1, ops2, iii, iim)
    return _tcd(m, aggW, aggb)
